# Initial kernel scaffold; baseline (speedup 1.0000x reference)
import functools

import jax
import jax.numpy as jnp
from jax import lax
from jax.experimental import pallas as pl
from jax.experimental.pallas import tpu as pltpu
from jax.experimental.pallas import tpu_sc as plsc

N = 10000
E = 320000
H = 128
NUM_GATES = 9
HAUG = 144

NC = 2
NS = 16
NW = NC * NS
EPW = E // NW
C = 80
NCHUNK = EPW // C
RPT = N // NS
ZR = 125

_mesh = plsc.VectorSubcoreMesh(core_axis_name="c", subcore_axis_name="s",
                               num_cores=NC, num_subcores=NS)


def _zero_zbuf(zbuf, width):
    def zrow(r, _):
        for c in range(width // 16):
            zbuf[r, pl.ds(c * 16, 16)] = jnp.zeros((16,), jnp.float32)
        return 0
    lax.fori_loop(0, ZR, zrow, 0)


def _zero_accum_slice(zbuf, accum, sid):
    def zcopy(j, _):
        pltpu.sync_copy(zbuf, accum.at[pl.ds(sid * RPT + j * ZR, ZR)])
        return 0
    lax.fori_loop(0, RPT // ZR, zcopy, 0)


def _write_out(zbuf, accum, out_hbm, cid, sid):
    def ocopy(j, _):
        pltpu.sync_copy(accum.at[pl.ds(sid * RPT + j * ZR, ZR)], zbuf)
        pltpu.sync_copy(zbuf, out_hbm.at[pl.ds(cid * N + sid * RPT + j * ZR, ZR)])
        return 0
    lax.fori_loop(0, RPT // ZR, ocopy, 0)


def _make_sc_segsum(width):
    @functools.partial(
        pl.kernel, mesh=_mesh,
        out_type=jax.ShapeDtypeStruct((2 * N, width), jnp.float32),
        scratch_types=[
            pltpu.VMEM((C,), jnp.int32),
            pltpu.VMEM((C,), jnp.int32),
            pltpu.VMEM((C, width), jnp.float32),
            pltpu.VMEM((ZR, width), jnp.float32),
            pltpu.VMEM_SHARED((N, width), jnp.float32),
            pltpu.SemaphoreType.DMA,
        ],
    )
    def k(x_hbm, src_hbm, dst_hbm, out_hbm, sidx, didx, rows, zbuf, accum, sem):
        cid = lax.axis_index("c")
        sid = lax.axis_index("s")
        wid = sid * NC + cid
        _zero_zbuf(zbuf, width)
        _zero_accum_slice(zbuf, accum, sid)
        plsc.subcore_barrier()
        ebase = wid * EPW

        def chunk(i, _):
            base = ebase + i * C
            pltpu.sync_copy(src_hbm.at[pl.ds(base, C)], sidx)
            pltpu.sync_copy(dst_hbm.at[pl.ds(base, C)], didx)
            pltpu.async_copy(x_hbm.at[sidx], rows, sem).wait()
            pltpu.sync_copy(rows, accum.at[didx], add=True)
            return 0
        lax.fori_loop(0, NCHUNK, chunk, 0)
        plsc.subcore_barrier()
        _write_out(zbuf, accum, out_hbm, cid, sid)

    return k


_sc_segsum_aug = _make_sc_segsum(HAUG)
_sc_segsum = _make_sc_segsum(H)


@functools.partial(
    pl.kernel, mesh=_mesh,
    out_type=jax.ShapeDtypeStruct((2 * N, HAUG), jnp.float32),
    scratch_types=[
        pltpu.VMEM((C,), jnp.int32),
        pltpu.VMEM((C,), jnp.int32),
        pltpu.VMEM((C, HAUG), jnp.float32),
        pltpu.VMEM((C,), jnp.float32),
        pltpu.VMEM((N,), jnp.float32),
        pltpu.VMEM((N,), jnp.float32),
        pltpu.VMEM((ZR, HAUG), jnp.float32),
        pltpu.VMEM_SHARED((N, HAUG), jnp.float32),
        pltpu.SemaphoreType.DMA,
    ],
)
def _sc_gat(haug_hbm, asrc_hbm, adst_hbm, src_hbm, dst_hbm, out_hbm,
            sidx, didx, rows, wbuf, asv, adv, zbuf, accum, sem):
    cid = lax.axis_index("c")
    sid = lax.axis_index("s")
    wid = sid * NC + cid
    _zero_zbuf(zbuf, HAUG)
    _zero_accum_slice(zbuf, accum, sid)
    pltpu.sync_copy(asrc_hbm, asv)
    pltpu.sync_copy(adst_hbm, adv)
    plsc.subcore_barrier()
    ebase = wid * EPW

    def chunk(i, _):
        base = ebase + i * C
        pltpu.sync_copy(src_hbm.at[pl.ds(base, C)], sidx)
        pltpu.sync_copy(dst_hbm.at[pl.ds(base, C)], didx)
        cp = pltpu.async_copy(haug_hbm.at[sidx], rows, sem)

        def wgrp(j, _):
            iv = sidx[pl.ds(j * 16, 16)]
            jv = didx[pl.ds(j * 16, 16)]
            s = plsc.load_gather(asv, [iv]) + plsc.load_gather(adv, [jv])
            wbuf[pl.ds(j * 16, 16)] = jnp.exp(jnp.maximum(s, 0.2 * s))
            return 0
        lax.fori_loop(0, C // 16, wgrp, 0)
        cp.wait()

        def scale(r, _):
            wr = wbuf[r]
            for c in range(HAUG // 16):
                sl = pl.ds(c * 16, 16)
                rows[r, sl] = rows[r, sl] * wr
            return 0
        lax.fori_loop(0, C, scale, 0)
        pltpu.sync_copy(rows, accum.at[didx], add=True)
        return 0
    lax.fori_loop(0, NCHUNK, chunk, 0)
    plsc.subcore_barrier()
    _write_out(zbuf, accum, out_hbm, cid, sid)


@functools.partial(
    pl.kernel, mesh=_mesh,
    out_type=jax.ShapeDtypeStruct((E, H), jnp.float32),
    scratch_types=[
        pltpu.VMEM((C,), jnp.int32),
        pltpu.VMEM((C,), jnp.int32),
        pltpu.VMEM((C, H), jnp.float32),
        pltpu.VMEM((C, H), jnp.float32),
        pltpu.SemaphoreType.DMA,
        pltpu.SemaphoreType.DMA,
    ],
)
def _sc_edge(y_hbm, src_hbm, dst_hbm, z_hbm, sidx, didx, rows_s, rows_d, sem1, sem2):
    cid = lax.axis_index("c")
    sid = lax.axis_index("s")
    wid = sid * NC + cid
    ebase = wid * EPW

    def chunk(i, _):
        base = ebase + i * C
        pltpu.sync_copy(src_hbm.at[pl.ds(base, C)], sidx)
        pltpu.sync_copy(dst_hbm.at[pl.ds(base, C)], didx)
        cp1 = pltpu.async_copy(y_hbm.at[sidx], rows_s, sem1)
        cp2 = pltpu.async_copy(y_hbm.at[didx], rows_d, sem2)
        cp1.wait()
        cp2.wait()

        def add(r, _):
            for c in range(H // 16):
                sl = pl.ds(c * 16, 16)
                rows_s[r, sl] = jnp.maximum(rows_s[r, sl] + rows_d[r, sl], 0.0)
            return 0
        lax.fori_loop(0, C, add, 0)
        pltpu.sync_copy(rows_s, z_hbm.at[pl.ds(base, C)])
        return 0
    lax.fori_loop(0, NCHUNK, chunk, 0)


def _ones_col(n):
    c16 = lax.broadcasted_iota(jnp.int32, (n, HAUG - H), 1)
    return jnp.where(c16 == 0, 1.0, 0.0).astype(jnp.float32)


def _tc_build_body(dims_r, gidx_r, wdt_r, bd_r, emb_r, out_r):
    g = gidx_r[:]
    oh = (g == lax.broadcasted_iota(jnp.int32, (1, NUM_GATES), 1)).astype(jnp.float32)
    out_r[:, 0:H // 2] = dims_r[:] * wdt_r[:] + bd_r[:]
    out_r[:, H // 2:H] = jnp.dot(oh, emb_r[:], preferred_element_type=jnp.float32)
    out_r[:, H:HAUG] = _ones_col(N)


def _tc_build(dims, gidx2, wdt, bd, emb):
    return pl.pallas_call(
        _tc_build_body,
        out_shape=jax.ShapeDtypeStruct((N, HAUG), jnp.float32),
    )(dims, gidx2, wdt, bd, emb)


def _tc_sage0_body(acc_r, x_r, wlt_r, bl_r, wrt_r, xout_r, invc_r):
    a = acc_r[0:N, :] + acc_r[N:2 * N, :]
    invc = 1.0 / jnp.maximum(a[:, H:H + 1], 1.0)
    mean = a[:, 0:H] * invc
    x = x_r[:, 0:H]
    z = (jnp.dot(mean, wlt_r[:], preferred_element_type=jnp.float32) + bl_r[:]
         + jnp.dot(x, wrt_r[:], preferred_element_type=jnp.float32))
    xout_r[:] = jnp.maximum(z, 0.0)
    invc_r[:] = invc


def _tc_sage0(acc, xaug, wlt, bl, wrt):
    return pl.pallas_call(
        _tc_sage0_body,
        out_shape=(jax.ShapeDtypeStruct((N, H), jnp.float32),
                   jax.ShapeDtypeStruct((N, 1), jnp.float32)),
    )(acc, xaug, wlt, bl, wrt)


def _tc_sage_body(acc_r, x_r, invc_r, wlt_r, bl_r, wrt_r, xout_r):
    a = acc_r[0:N, :] + acc_r[N:2 * N, :]
    mean = a * invc_r[:]
    z = (jnp.dot(mean, wlt_r[:], preferred_element_type=jnp.float32) + bl_r[:]
         + jnp.dot(x_r[:], wrt_r[:], preferred_element_type=jnp.float32))
    xout_r[:] = jnp.maximum(z, 0.0)


def _tc_sage(acc, x, invc, wlt, bl, wrt):
    return pl.pallas_call(
        _tc_sage_body,
        out_shape=jax.ShapeDtypeStruct((N, H), jnp.float32),
    )(acc, x, invc, wlt, bl, wrt)


def _tc_gatprep_body(x_r, gwt_r, asc_r, adc_r, haug_r, asr_r, adr_r):
    h = jnp.dot(x_r[:], gwt_r[:], preferred_element_type=jnp.float32)
    haug_r[:, 0:H] = h
    haug_r[:, H:HAUG] = _ones_col(N)
    asr_r[:] = jnp.dot(h, asc_r[:], preferred_element_type=jnp.float32)
    adr_r[:] = jnp.dot(h, adc_r[:], preferred_element_type=jnp.float32)


def _tc_gatprep(x, gwt, asc, adc):
    return pl.pallas_call(
        _tc_gatprep_body,
        out_shape=(jax.ShapeDtypeStruct((N, HAUG), jnp.float32),
                   jax.ShapeDtypeStruct((N, 1), jnp.float32),
                   jax.ShapeDtypeStruct((N, 1), jnp.float32)),
    )(x, gwt, asc, adc)


def _tc_gatfin_body(acc_r, haug_r, asr_r, adr_r, gb_r, e0wt_r, e0bh_r, y_r):
    a = acc_r[0:N, :] + acc_r[N:2 * N, :]
    wh = a[:, 0:H]
    den = a[:, H:H + 1]
    h = haug_r[:, 0:H]
    s = asr_r[:] + adr_r[:]
    wl = jnp.exp(jnp.maximum(s, 0.2 * s))
    x4 = jnp.maximum((wh + wl * h) / (den + wl + 1e-16) + gb_r[:], 0.0)
    y_r[:] = jnp.dot(x4, e0wt_r[:], preferred_element_type=jnp.float32) + e0bh_r[:]


def _tc_gatfin(acc, haug, asr, adr, gb, e0wt, e0bh):
    return pl.pallas_call(
        _tc_gatfin_body,
        out_shape=jax.ShapeDtypeStruct((N, H), jnp.float32),
    )(acc, haug, asr, adr, gb, e0wt, e0bh)


BE = 2000


def _tc_final_body(z_r, e1wt_r, e1b_r, owt_r, ob_r, out_r):
    t = jnp.maximum(jnp.dot(z_r[:], e1wt_r[:], preferred_element_type=jnp.float32)
                    + e1b_r[:], 0.0)
    out_r[:] = jnp.dot(t, owt_r[:], preferred_element_type=jnp.float32) + ob_r[:]


def _tc_final(z, e1wt, e1b, owt, ob):
    grid = (E // BE,)
    return pl.pallas_call(
        _tc_final_body,
        grid=grid,
        in_specs=[
            pl.BlockSpec((BE, H), lambda i: (i, 0)),
            pl.BlockSpec((H, H), lambda i: (0, 0)),
            pl.BlockSpec((1, H), lambda i: (0, 0)),
            pl.BlockSpec((H, 1), lambda i: (0, 0)),
            pl.BlockSpec((1, 1), lambda i: (0, 0)),
        ],
        out_specs=pl.BlockSpec((BE, 1), lambda i: (i, 0)),
        out_shape=jax.ShapeDtypeStruct((E, 1), jnp.float32),
    )(z, e1wt, e1b, owt, ob)


def kernel(dims, gate_indices, edge_index, emb_table, W_dim, b_dim,
           sage0_Wl, sage0_bl, sage0_Wr, sage1_Wl, sage1_bl, sage1_Wr,
           sage2_Wl, sage2_bl, sage2_Wr,
           gat_W, gat_att_src, gat_att_dst, gat_b,
           e0_W, e0_b, e1_W, e1_b, out_W, out_b):
    f32 = jnp.float32
    src = edge_index[0].astype(jnp.int32)
    dst = edge_index[1].astype(jnp.int32)
    gidx2 = gate_indices.astype(jnp.int32).reshape(N, 1)

    xaug0 = _tc_build(dims.astype(f32), gidx2, W_dim.reshape(1, H // 2),
                      b_dim.reshape(1, H // 2), emb_table)

    acc0 = _sc_segsum_aug(xaug0, src, dst)
    x1, invc = _tc_sage0(acc0, xaug0, sage0_Wl.T, sage0_bl.reshape(1, H), sage0_Wr.T)

    acc1 = _sc_segsum(x1, src, dst)
    x2 = _tc_sage(acc1, x1, invc, sage1_Wl.T, sage1_bl.reshape(1, H), sage1_Wr.T)

    acc2 = _sc_segsum(x2, src, dst)
    x3 = _tc_sage(acc2, x2, invc, sage2_Wl.T, sage2_bl.reshape(1, H), sage2_Wr.T)

    haug, asr, adr = _tc_gatprep(x3, gat_W.T, gat_att_src.reshape(H, 1),
                                 gat_att_dst.reshape(H, 1))
    accg = _sc_gat(haug, asr.reshape(N), adr.reshape(N), src, dst)
    y = _tc_gatfin(accg, haug, asr, adr, gat_b.reshape(1, H), e0_W.T,
                   (0.5 * e0_b).reshape(1, H))

    z = _sc_edge(y, src, dst)
    return _tc_final(z, e1_W.T, e1_b.reshape(1, H), out_W.T, out_b.reshape(1, 1))


# speedup vs baseline: 7.7988x; 7.7988x over previous
import functools

import jax
import jax.numpy as jnp
from jax import lax
from jax.experimental import pallas as pl
from jax.experimental.pallas import tpu as pltpu
from jax.experimental.pallas import tpu_sc as plsc

N = 10000
E = 320000
H = 128
NUM_GATES = 9

NC = 2
NS = 16
NW = NC * NS
EPW = E // NW
C = 80
NCHUNK = EPW // C
NPAD = 10240
RPT = NPAD // NS
ZR = 80

_mesh = plsc.VectorSubcoreMesh(core_axis_name="c", subcore_axis_name="s",
                               num_cores=NC, num_subcores=NS)
_sc_params = pltpu.CompilerParams(needs_layout_passes=False)


def _zero_rows(rows):
    def zrow(r, _):
        for c in range(H // 16):
            rows[r, pl.ds(c * 16, 16)] = jnp.zeros((16,), jnp.float32)
        return 0
    lax.fori_loop(0, ZR, zrow, 0)


def _zero_vec(v):
    def zrow(r, _):
        v[pl.ds(r * 16, 16)] = jnp.zeros((16,), jnp.float32)
        return 0
    lax.fori_loop(0, N // 16, zrow, 0)


def _zero_accum_slice(rows, accum, sid):
    def zcopy(j, _):
        pltpu.sync_copy(rows.at[pl.ds(0, ZR)], accum.at[pl.ds(sid * RPT + j * ZR, ZR)])
        return 0
    lax.fori_loop(0, RPT // ZR, zcopy, 0)


def _write_out(rows, accum, out_hbm, cid, sid):
    def ocopy(j, _):
        pltpu.sync_copy(accum.at[pl.ds(sid * RPT + j * ZR, ZR)], rows.at[pl.ds(0, ZR)])
        pltpu.sync_copy(rows.at[pl.ds(0, ZR)], out_hbm.at[pl.ds(cid * NPAD + sid * RPT + j * ZR, ZR)])
        return 0
    lax.fori_loop(0, RPT // ZR, ocopy, 0)


@functools.partial(
    pl.kernel, mesh=_mesh, compiler_params=_sc_params,
    out_type=(jax.ShapeDtypeStruct((2 * NPAD, H), jnp.float32),
              jax.ShapeDtypeStruct((NW * N,), jnp.float32)),
    scratch_types=[
        pltpu.VMEM((C,), jnp.int32),
        pltpu.VMEM((C,), jnp.int32),
        pltpu.VMEM((C, H), jnp.float32),
        pltpu.VMEM((N,), jnp.float32),
        pltpu.VMEM_SHARED((NPAD, H), jnp.float32),
        pltpu.SemaphoreType.DMA,
    ],
)
def _sc_segsum_cnt(x_hbm, src_hbm, dst_hbm, out_hbm, cnt_hbm,
                   sidx, didx, rows, cnt_v, accum, sem):
    cid = lax.axis_index("c")
    sid = lax.axis_index("s")
    wid = sid * NC + cid
    _zero_rows(rows)
    _zero_accum_slice(rows, accum, sid)
    _zero_vec(cnt_v)
    plsc.subcore_barrier()
    ebase = wid * EPW
    ones16 = jnp.full((16,), 1.0, jnp.float32)

    def chunk(i, _):
        base = ebase + i * C
        pltpu.sync_copy(src_hbm.at[pl.ds(base, C)], sidx)
        pltpu.sync_copy(dst_hbm.at[pl.ds(base, C)], didx)
        cp = pltpu.async_copy(x_hbm.at[sidx], rows, sem)

        def cgrp(j, _):
            jv = didx[pl.ds(j * 16, 16)]
            plsc.addupdate_scatter(cnt_v, [jv], ones16)
            return 0
        lax.fori_loop(0, C // 16, cgrp, 0)
        cp.wait()
        pltpu.sync_copy(rows, accum.at[didx], add=True)
        return 0
    lax.fori_loop(0, NCHUNK, chunk, 0)
    plsc.subcore_barrier()
    _write_out(rows, accum, out_hbm, cid, sid)
    pltpu.sync_copy(cnt_v, cnt_hbm.at[pl.ds(wid * N, N)])


@functools.partial(
    pl.kernel, mesh=_mesh, compiler_params=_sc_params,
    out_type=jax.ShapeDtypeStruct((2 * NPAD, H), jnp.float32),
    scratch_types=[
        pltpu.VMEM((C,), jnp.int32),
        pltpu.VMEM((C,), jnp.int32),
        pltpu.VMEM((C, H), jnp.float32),
        pltpu.VMEM_SHARED((NPAD, H), jnp.float32),
        pltpu.SemaphoreType.DMA,
    ],
)
def _sc_segsum(x_hbm, src_hbm, dst_hbm, out_hbm, sidx, didx, rows, accum, sem):
    cid = lax.axis_index("c")
    sid = lax.axis_index("s")
    wid = sid * NC + cid
    _zero_rows(rows)
    _zero_accum_slice(rows, accum, sid)
    plsc.subcore_barrier()
    ebase = wid * EPW

    def chunk(i, _):
        base = ebase + i * C
        pltpu.sync_copy(src_hbm.at[pl.ds(base, C)], sidx)
        pltpu.sync_copy(dst_hbm.at[pl.ds(base, C)], didx)
        pltpu.async_copy(x_hbm.at[sidx], rows, sem).wait()
        pltpu.sync_copy(rows, accum.at[didx], add=True)
        return 0
    lax.fori_loop(0, NCHUNK, chunk, 0)
    plsc.subcore_barrier()
    _write_out(rows, accum, out_hbm, cid, sid)


@functools.partial(
    pl.kernel, mesh=_mesh, compiler_params=_sc_params,
    out_type=(jax.ShapeDtypeStruct((2 * NPAD, H), jnp.float32),
              jax.ShapeDtypeStruct((NW * N,), jnp.float32)),
    scratch_types=[
        pltpu.VMEM((C,), jnp.int32),
        pltpu.VMEM((C,), jnp.int32),
        pltpu.VMEM((C, H), jnp.float32),
        pltpu.VMEM((C,), jnp.float32),
        pltpu.VMEM((N,), jnp.float32),
        pltpu.VMEM((N,), jnp.float32),
        pltpu.VMEM((N,), jnp.float32),
        pltpu.VMEM_SHARED((NPAD, H), jnp.float32),
        pltpu.SemaphoreType.DMA,
    ],
)
def _sc_gat(h_hbm, asrc_hbm, adst_hbm, src_hbm, dst_hbm, out_hbm, den_hbm,
            sidx, didx, rows, wbuf, asv, adv, den_v, accum, sem):
    cid = lax.axis_index("c")
    sid = lax.axis_index("s")
    wid = sid * NC + cid
    _zero_rows(rows)
    _zero_accum_slice(rows, accum, sid)
    _zero_vec(den_v)
    pltpu.sync_copy(asrc_hbm, asv)
    pltpu.sync_copy(adst_hbm, adv)
    plsc.subcore_barrier()
    ebase = wid * EPW

    def chunk(i, _):
        base = ebase + i * C
        pltpu.sync_copy(src_hbm.at[pl.ds(base, C)], sidx)
        pltpu.sync_copy(dst_hbm.at[pl.ds(base, C)], didx)
        cp = pltpu.async_copy(h_hbm.at[sidx], rows, sem)

        def wgrp(j, _):
            iv = sidx[pl.ds(j * 16, 16)]
            jv = didx[pl.ds(j * 16, 16)]
            s = plsc.load_gather(asv, [iv]) + plsc.load_gather(adv, [jv])
            w = jnp.exp(jnp.maximum(s, 0.2 * s))
            wbuf[pl.ds(j * 16, 16)] = w
            plsc.addupdate_scatter(den_v, [jv], w)
            return 0
        lax.fori_loop(0, C // 16, wgrp, 0)
        cp.wait()

        def scale(r, _):
            wr = plsc.load_gather(wbuf, [jnp.zeros((16,), jnp.int32) + r])
            for c in range(H // 16):
                sl = pl.ds(c * 16, 16)
                rows[r, sl] = rows[r, sl] * wr
            return 0
        lax.fori_loop(0, C, scale, 0)
        pltpu.sync_copy(rows, accum.at[didx], add=True)
        return 0
    lax.fori_loop(0, NCHUNK, chunk, 0)
    plsc.subcore_barrier()
    _write_out(rows, accum, out_hbm, cid, sid)
    pltpu.sync_copy(den_v, den_hbm.at[pl.ds(wid * N, N)])


@functools.partial(
    pl.kernel, mesh=_mesh, compiler_params=_sc_params,
    out_type=jax.ShapeDtypeStruct((E, H), jnp.float32),
    scratch_types=[
        pltpu.VMEM((C,), jnp.int32),
        pltpu.VMEM((C,), jnp.int32),
        pltpu.VMEM((C, H), jnp.float32),
        pltpu.VMEM((C, H), jnp.float32),
        pltpu.SemaphoreType.DMA,
        pltpu.SemaphoreType.DMA,
    ],
)
def _sc_edge(y_hbm, src_hbm, dst_hbm, z_hbm, sidx, didx, rows_s, rows_d, sem1, sem2):
    cid = lax.axis_index("c")
    sid = lax.axis_index("s")
    wid = sid * NC + cid
    ebase = wid * EPW

    def chunk(i, _):
        base = ebase + i * C
        pltpu.sync_copy(src_hbm.at[pl.ds(base, C)], sidx)
        pltpu.sync_copy(dst_hbm.at[pl.ds(base, C)], didx)
        cp1 = pltpu.async_copy(y_hbm.at[sidx], rows_s, sem1)
        cp2 = pltpu.async_copy(y_hbm.at[didx], rows_d, sem2)
        cp1.wait()
        cp2.wait()

        def add(r, _):
            for c in range(H // 16):
                sl = pl.ds(c * 16, 16)
                rows_s[r, sl] = jnp.maximum(rows_s[r, sl] + rows_d[r, sl], 0.0)
            return 0
        lax.fori_loop(0, C, add, 0)
        pltpu.sync_copy(rows_s, z_hbm.at[pl.ds(base, C)])
        return 0
    lax.fori_loop(0, NCHUNK, chunk, 0)


_ONES32 = None


def _col_sum(p2d_r):
    ones = jnp.full((NW, 1), 1.0, jnp.float32)
    return lax.dot_general(p2d_r[:], ones, (((0,), (0,)), ((), ())),
                           preferred_element_type=jnp.float32)


def _tc_build_body(dims_r, gidx_r, wdt_r, bd_r, emb_r, out_r):
    g = gidx_r[:]
    oh = (g == lax.broadcasted_iota(jnp.int32, (1, NUM_GATES), 1)).astype(jnp.float32)
    out_r[:, 0:H // 2] = dims_r[:] * wdt_r[:] + bd_r[:]
    out_r[:, H // 2:H] = jnp.dot(oh, emb_r[:], preferred_element_type=jnp.float32)


def _tc_build(dims, gidx2, wdt, bd, emb):
    return pl.pallas_call(
        _tc_build_body,
        out_shape=jax.ShapeDtypeStruct((N, H), jnp.float32),
    )(dims, gidx2, wdt, bd, emb)


def _tc_sage0_body(acc_r, cntp_r, x_r, wlt_r, bl_r, wrt_r, xout_r, invc_r):
    a = acc_r[0:N, :] + acc_r[NPAD:NPAD + N, :]
    invc = 1.0 / jnp.maximum(_col_sum(cntp_r), 1.0)
    mean = a * invc
    z = (jnp.dot(mean, wlt_r[:], preferred_element_type=jnp.float32) + bl_r[:]
         + jnp.dot(x_r[:], wrt_r[:], preferred_element_type=jnp.float32))
    xout_r[:] = jnp.maximum(z, 0.0)
    invc_r[:] = invc


def _tc_sage0(acc, cntp, x, wlt, bl, wrt):
    return pl.pallas_call(
        _tc_sage0_body,
        out_shape=(jax.ShapeDtypeStruct((N, H), jnp.float32),
                   jax.ShapeDtypeStruct((N, 1), jnp.float32)),
    )(acc, cntp, x, wlt, bl, wrt)


def _tc_sage_body(acc_r, x_r, invc_r, wlt_r, bl_r, wrt_r, xout_r):
    a = acc_r[0:N, :] + acc_r[NPAD:NPAD + N, :]
    mean = a * invc_r[:]
    z = (jnp.dot(mean, wlt_r[:], preferred_element_type=jnp.float32) + bl_r[:]
         + jnp.dot(x_r[:], wrt_r[:], preferred_element_type=jnp.float32))
    xout_r[:] = jnp.maximum(z, 0.0)


def _tc_sage(acc, x, invc, wlt, bl, wrt):
    return pl.pallas_call(
        _tc_sage_body,
        out_shape=jax.ShapeDtypeStruct((N, H), jnp.float32),
    )(acc, x, invc, wlt, bl, wrt)


def _tc_gatprep_body(x_r, gwt_r, asc_r, adc_r, h_r, asr_r, adr_r):
    h = jnp.dot(x_r[:], gwt_r[:], preferred_element_type=jnp.float32)
    h_r[:] = h
    asr_r[:] = jnp.dot(h, asc_r[:], preferred_element_type=jnp.float32)
    adr_r[:] = jnp.dot(h, adc_r[:], preferred_element_type=jnp.float32)


def _tc_gatprep(x, gwt, asc, adc):
    return pl.pallas_call(
        _tc_gatprep_body,
        out_shape=(jax.ShapeDtypeStruct((N, H), jnp.float32),
                   jax.ShapeDtypeStruct((N, 1), jnp.float32),
                   jax.ShapeDtypeStruct((N, 1), jnp.float32)),
    )(x, gwt, asc, adc)


def _tc_gatfin_body(acc_r, denp_r, h_r, asr_r, adr_r, gb_r, e0wt_r, e0bh_r, y_r):
    wh = acc_r[0:N, :] + acc_r[NPAD:NPAD + N, :]
    den = _col_sum(denp_r)
    h = h_r[:]
    s = asr_r[:] + adr_r[:]
    wl = jnp.exp(jnp.maximum(s, 0.2 * s))
    x4 = jnp.maximum((wh + wl * h) / (den + wl + 1e-16) + gb_r[:], 0.0)
    y_r[:] = jnp.dot(x4, e0wt_r[:], preferred_element_type=jnp.float32) + e0bh_r[:]


def _tc_gatfin(acc, denp, h, asr, adr, gb, e0wt, e0bh):
    return pl.pallas_call(
        _tc_gatfin_body,
        out_shape=jax.ShapeDtypeStruct((N, H), jnp.float32),
    )(acc, denp, h, asr, adr, gb, e0wt, e0bh)


BE = 2000


def _tc_final_body(z_r, e1wt_r, e1b_r, owt_r, ob_r, out_r):
    t = jnp.maximum(jnp.dot(z_r[:], e1wt_r[:], preferred_element_type=jnp.float32)
                    + e1b_r[:], 0.0)
    out_r[:] = jnp.dot(t, owt_r[:], preferred_element_type=jnp.float32) + ob_r[:]


def _tc_final(z, e1wt, e1b, owt, ob):
    grid = (E // BE,)
    return pl.pallas_call(
        _tc_final_body,
        grid=grid,
        in_specs=[
            pl.BlockSpec((BE, H), lambda i: (i, 0)),
            pl.BlockSpec((H, H), lambda i: (0, 0)),
            pl.BlockSpec((1, H), lambda i: (0, 0)),
            pl.BlockSpec((H, 1), lambda i: (0, 0)),
            pl.BlockSpec((1, 1), lambda i: (0, 0)),
        ],
        out_specs=pl.BlockSpec((BE, 1), lambda i: (i, 0)),
        out_shape=jax.ShapeDtypeStruct((E, 1), jnp.float32),
    )(z, e1wt, e1b, owt, ob)


def kernel(dims, gate_indices, edge_index, emb_table, W_dim, b_dim,
           sage0_Wl, sage0_bl, sage0_Wr, sage1_Wl, sage1_bl, sage1_Wr,
           sage2_Wl, sage2_bl, sage2_Wr,
           gat_W, gat_att_src, gat_att_dst, gat_b,
           e0_W, e0_b, e1_W, e1_b, out_W, out_b):
    f32 = jnp.float32
    src = edge_index[0].astype(jnp.int32)
    dst = edge_index[1].astype(jnp.int32)
    gidx2 = gate_indices.astype(jnp.int32).reshape(N, 1)

    x0 = _tc_build(dims.astype(f32), gidx2, W_dim.reshape(1, H // 2),
                   b_dim.reshape(1, H // 2), emb_table)

    acc0, cntp = _sc_segsum_cnt(x0, src, dst)
    x1, invc = _tc_sage0(acc0, cntp.reshape(NW, N), x0,
                         sage0_Wl.T, sage0_bl.reshape(1, H), sage0_Wr.T)

    acc1 = _sc_segsum(x1, src, dst)
    x2 = _tc_sage(acc1, x1, invc, sage1_Wl.T, sage1_bl.reshape(1, H), sage1_Wr.T)

    acc2 = _sc_segsum(x2, src, dst)
    x3 = _tc_sage(acc2, x2, invc, sage2_Wl.T, sage2_bl.reshape(1, H), sage2_Wr.T)

    h, asr, adr = _tc_gatprep(x3, gat_W.T, gat_att_src.reshape(H, 1),
                              gat_att_dst.reshape(H, 1))
    accg, denp = _sc_gat(h, asr.reshape(N), adr.reshape(N), src, dst)
    y = _tc_gatfin(accg, denp.reshape(NW, N), h, asr, adr,
                   gat_b.reshape(1, H), e0_W.T, (0.5 * e0_b).reshape(1, H))

    z = _sc_edge(y, src, dst)
    return _tc_final(z, e1_W.T, e1_b.reshape(1, H), out_W.T, out_b.reshape(1, 1))


# speedup vs baseline: 8.8713x; 1.1375x over previous
import functools

import jax
import jax.numpy as jnp
from jax import lax
from jax.experimental import pallas as pl
from jax.experimental.pallas import tpu as pltpu
from jax.experimental.pallas import tpu_sc as plsc

N = 10000
E = 320000
H = 128
NUM_GATES = 9

NC = 2
NS = 16
NW = NC * NS
EPW = E // NW
C = 80
NCHUNK = EPW // C
NB = 2
NITER = (NCHUNK + NB - 1) // NB
NPAD = 10240
RPT = NPAD // NS
ZR = 80

_mesh = plsc.VectorSubcoreMesh(core_axis_name="c", subcore_axis_name="s",
                               num_cores=NC, num_subcores=NS)
_sc_params = pltpu.CompilerParams(needs_layout_passes=False)


def _zero_rows(rows):
    def zrow(r, _):
        for c in range(H // 16):
            rows[r, pl.ds(c * 16, 16)] = jnp.zeros((16,), jnp.float32)
        return 0
    lax.fori_loop(0, ZR, zrow, 0)


def _zero_vec(v):
    def zrow(r, _):
        v[pl.ds(r * 16, 16)] = jnp.zeros((16,), jnp.float32)
        return 0
    lax.fori_loop(0, N // 16, zrow, 0)


def _zero_accum_slice(rows, accum, sid):
    def zcopy(j, _):
        pltpu.sync_copy(rows.at[pl.ds(0, ZR)], accum.at[pl.ds(sid * RPT + j * ZR, ZR)])
        return 0
    lax.fori_loop(0, RPT // ZR, zcopy, 0)


def _write_out(rows, accum, out_hbm, cid, sid):
    def ocopy(j, _):
        pltpu.sync_copy(accum.at[pl.ds(sid * RPT + j * ZR, ZR)], rows.at[pl.ds(0, ZR)])
        pltpu.sync_copy(rows.at[pl.ds(0, ZR)],
                        out_hbm.at[pl.ds(cid * NPAD + sid * RPT + j * ZR, ZR)])
        return 0
    lax.fori_loop(0, RPT // ZR, ocopy, 0)


def _segsum_pipeline(x_hbm, src_hbm, dst_hbm, accum, ebase,
                     sidx, didx, rows, gsem, ssem, on_chunk=None):
    def prep(k, b):
        base = ebase + k * C
        pltpu.sync_copy(src_hbm.at[pl.ds(base, C)], sidx[b])
        pltpu.sync_copy(dst_hbm.at[pl.ds(base, C)], didx[b])
        pltpu.async_copy(x_hbm.at[sidx[b]], rows[b], gsem[b])

    for b in range(NB):
        prep(b, b)

    def outer(g, _):
        for b in range(NB):
            k = NB * g + b

            @pl.when(k < NCHUNK)
            def _step(b=b, k=k):
                pltpu.make_async_copy(x_hbm.at[sidx[b]], rows[b], gsem[b]).wait()
                if on_chunk is not None:
                    on_chunk(k, b)
                pltpu.async_copy(rows[b], accum.at[didx[b]], ssem[b], add=True)

                @pl.when(k + NB < NCHUNK)
                def _prep_next(b=b, k=k):
                    pltpu.make_async_copy(rows[b], accum.at[didx[b]], ssem[b]).wait()
                    prep(k + NB, b)
        return 0
    lax.fori_loop(0, NITER, outer, 0)
    for b in range(NB):
        pltpu.make_async_copy(rows[b], accum.at[didx[b]], ssem[b]).wait()


_SEG_SCRATCH = [
    pltpu.VMEM((C,), jnp.int32), pltpu.VMEM((C,), jnp.int32),
    pltpu.VMEM((C,), jnp.int32), pltpu.VMEM((C,), jnp.int32),
    pltpu.VMEM((C, H), jnp.float32), pltpu.VMEM((C, H), jnp.float32),
    pltpu.VMEM_SHARED((NPAD, H), jnp.float32),
    pltpu.SemaphoreType.DMA, pltpu.SemaphoreType.DMA,
    pltpu.SemaphoreType.DMA, pltpu.SemaphoreType.DMA,
]


@functools.partial(
    pl.kernel, mesh=_mesh, compiler_params=_sc_params,
    out_type=(jax.ShapeDtypeStruct((2 * NPAD, H), jnp.float32),
              jax.ShapeDtypeStruct((NW * N,), jnp.float32)),
    scratch_types=_SEG_SCRATCH + [pltpu.VMEM((N,), jnp.float32)],
)
def _sc_segsum_cnt(x_hbm, src_hbm, dst_hbm, out_hbm, cnt_hbm,
                   sidx0, sidx1, didx0, didx1, rows0, rows1, accum,
                   gsem0, gsem1, ssem0, ssem1, cnt_v):
    cid = lax.axis_index("c")
    sid = lax.axis_index("s")
    wid = sid * NC + cid
    _zero_rows(rows0)
    _zero_accum_slice(rows0, accum, sid)
    _zero_vec(cnt_v)
    plsc.subcore_barrier()
    didx = (didx0, didx1)
    ones16 = jnp.full((16,), 1.0, jnp.float32)

    def on_chunk(k, b):
        def cgrp(j, _):
            jv = didx[b][pl.ds(j * 16, 16)]
            plsc.addupdate_scatter(cnt_v, [jv], ones16)
            return 0
        lax.fori_loop(0, C // 16, cgrp, 0)

    _segsum_pipeline(x_hbm, src_hbm, dst_hbm, accum, wid * EPW,
                     (sidx0, sidx1), didx, (rows0, rows1),
                     (gsem0, gsem1), (ssem0, ssem1), on_chunk)
    plsc.subcore_barrier()
    _write_out(rows0, accum, out_hbm, cid, sid)
    pltpu.sync_copy(cnt_v, cnt_hbm.at[pl.ds(wid * N, N)])


@functools.partial(
    pl.kernel, mesh=_mesh, compiler_params=_sc_params,
    out_type=jax.ShapeDtypeStruct((2 * NPAD, H), jnp.float32),
    scratch_types=_SEG_SCRATCH,
)
def _sc_segsum(x_hbm, src_hbm, dst_hbm, out_hbm,
               sidx0, sidx1, didx0, didx1, rows0, rows1, accum,
               gsem0, gsem1, ssem0, ssem1):
    cid = lax.axis_index("c")
    sid = lax.axis_index("s")
    wid = sid * NC + cid
    _zero_rows(rows0)
    _zero_accum_slice(rows0, accum, sid)
    plsc.subcore_barrier()
    _segsum_pipeline(x_hbm, src_hbm, dst_hbm, accum, wid * EPW,
                     (sidx0, sidx1), (didx0, didx1), (rows0, rows1),
                     (gsem0, gsem1), (ssem0, ssem1))
    plsc.subcore_barrier()
    _write_out(rows0, accum, out_hbm, cid, sid)


@functools.partial(
    pl.kernel, mesh=_mesh, compiler_params=_sc_params,
    out_type=(jax.ShapeDtypeStruct((E,), jnp.float32),
              jax.ShapeDtypeStruct((NW * N,), jnp.float32)),
    scratch_types=[
        pltpu.VMEM((EPW,), jnp.int32),
        pltpu.VMEM((EPW,), jnp.int32),
        pltpu.VMEM((EPW,), jnp.float32),
        pltpu.VMEM((N,), jnp.float32),
        pltpu.VMEM((N,), jnp.float32),
        pltpu.VMEM((N,), jnp.float32),
    ],
)
def _sc_gatw(asrc_hbm, adst_hbm, src_hbm, dst_hbm, w_hbm, den_hbm,
             sidx_all, didx_all, wall, asv, adv, den_v):
    cid = lax.axis_index("c")
    sid = lax.axis_index("s")
    wid = sid * NC + cid
    ebase = wid * EPW
    pltpu.sync_copy(src_hbm.at[pl.ds(ebase, EPW)], sidx_all)
    pltpu.sync_copy(dst_hbm.at[pl.ds(ebase, EPW)], didx_all)
    pltpu.sync_copy(asrc_hbm, asv)
    pltpu.sync_copy(adst_hbm, adv)
    _zero_vec(den_v)

    def grp(j, _):
        sl = pl.ds(j * 16, 16)
        iv = sidx_all[sl]
        jv = didx_all[sl]
        s = plsc.load_gather(asv, [iv]) + plsc.load_gather(adv, [jv])
        w = jnp.exp(jnp.maximum(s, 0.2 * s))
        wall[sl] = w
        plsc.addupdate_scatter(den_v, [jv], w)
        return 0
    lax.fori_loop(0, EPW // 16, grp, 0)
    pltpu.sync_copy(wall, w_hbm.at[pl.ds(ebase, EPW)])
    pltpu.sync_copy(den_v, den_hbm.at[pl.ds(wid * N, N)])


@functools.partial(
    pl.kernel, mesh=_mesh, compiler_params=_sc_params,
    out_type=jax.ShapeDtypeStruct((2 * NPAD, H), jnp.float32),
    scratch_types=_SEG_SCRATCH + [pltpu.VMEM((C,), jnp.float32),
                                  pltpu.VMEM((C,), jnp.float32)],
)
def _sc_gat(h_hbm, w_hbm, src_hbm, dst_hbm, out_hbm,
            sidx0, sidx1, didx0, didx1, rows0, rows1, accum,
            gsem0, gsem1, ssem0, ssem1, wc0, wc1):
    cid = lax.axis_index("c")
    sid = lax.axis_index("s")
    wid = sid * NC + cid
    ebase = wid * EPW
    _zero_rows(rows0)
    _zero_accum_slice(rows0, accum, sid)
    plsc.subcore_barrier()
    rows = (rows0, rows1)
    wc = (wc0, wc1)

    def on_chunk(k, b):
        pltpu.sync_copy(w_hbm.at[pl.ds(ebase + k * C, C)], wc[b])

        def scale(r, _):
            wr = plsc.load_gather(wc[b], [jnp.zeros((16,), jnp.int32) + r])
            for c in range(H // 16):
                sl = pl.ds(c * 16, 16)
                rows[b][r, sl] = rows[b][r, sl] * wr
            return 0
        lax.fori_loop(0, C, scale, 0)

    _segsum_pipeline(h_hbm, src_hbm, dst_hbm, accum, ebase,
                     (sidx0, sidx1), (didx0, didx1), rows,
                     (gsem0, gsem1), (ssem0, ssem1), on_chunk)
    plsc.subcore_barrier()
    _write_out(rows0, accum, out_hbm, cid, sid)


@functools.partial(
    pl.kernel, mesh=_mesh, compiler_params=_sc_params,
    out_type=jax.ShapeDtypeStruct((E, H), jnp.float32),
    scratch_types=[
        pltpu.VMEM((C,), jnp.int32), pltpu.VMEM((C,), jnp.int32),
        pltpu.VMEM((C,), jnp.int32), pltpu.VMEM((C,), jnp.int32),
        pltpu.VMEM((C, H), jnp.float32), pltpu.VMEM((C, H), jnp.float32),
        pltpu.VMEM((C, H), jnp.float32), pltpu.VMEM((C, H), jnp.float32),
        pltpu.SemaphoreType.DMA, pltpu.SemaphoreType.DMA,
        pltpu.SemaphoreType.DMA, pltpu.SemaphoreType.DMA,
        pltpu.SemaphoreType.DMA, pltpu.SemaphoreType.DMA,
    ],
)
def _sc_edge(y_hbm, src_hbm, dst_hbm, z_hbm,
             sidx0, sidx1, didx0, didx1, rs0, rs1, rd0, rd1,
             gs0, gs1, gd0, gd1, zsem0, zsem1):
    cid = lax.axis_index("c")
    sid = lax.axis_index("s")
    wid = sid * NC + cid
    ebase = wid * EPW
    sidx = (sidx0, sidx1)
    didx = (didx0, didx1)
    rs = (rs0, rs1)
    rd = (rd0, rd1)
    gs = (gs0, gs1)
    gd = (gd0, gd1)
    zsem = (zsem0, zsem1)

    def prep(k, b):
        base = ebase + k * C
        pltpu.sync_copy(src_hbm.at[pl.ds(base, C)], sidx[b])
        pltpu.sync_copy(dst_hbm.at[pl.ds(base, C)], didx[b])
        pltpu.async_copy(y_hbm.at[sidx[b]], rs[b], gs[b])
        pltpu.async_copy(y_hbm.at[didx[b]], rd[b], gd[b])

    for b in range(NB):
        prep(b, b)

    def outer(g, _):
        for b in range(NB):
            k = NB * g + b

            @pl.when(k < NCHUNK)
            def _step(b=b, k=k):
                base = ebase + k * C
                pltpu.make_async_copy(y_hbm.at[sidx[b]], rs[b], gs[b]).wait()
                pltpu.make_async_copy(y_hbm.at[didx[b]], rd[b], gd[b]).wait()

                def add(r, _):
                    for c in range(H // 16):
                        sl = pl.ds(c * 16, 16)
                        rs[b][r, sl] = jnp.maximum(rs[b][r, sl] + rd[b][r, sl], 0.0)
                    return 0
                lax.fori_loop(0, C, add, 0)
                pltpu.async_copy(rs[b], z_hbm.at[pl.ds(base, C)], zsem[b])

                @pl.when(k + NB < NCHUNK)
                def _prep_next(b=b, k=k):
                    pltpu.make_async_copy(rs[b], z_hbm.at[pl.ds(ebase, C)],
                                          zsem[b]).wait()
                    prep(k + NB, b)
        return 0
    lax.fori_loop(0, NITER, outer, 0)
    for b in range(NB):
        pltpu.make_async_copy(rs[b], z_hbm.at[pl.ds(ebase, C)], zsem[b]).wait()


BN = 2000
_HP = lax.Precision.HIGHEST


def _col_sum(p2d):
    ones = jnp.full((NW, 1), 1.0, jnp.float32)
    return lax.dot_general(p2d, ones, (((0,), (0,)), ((), ())),
                           preferred_element_type=jnp.float32, precision=_HP)


def _rows_spec(width):
    return pl.BlockSpec((BN, width), lambda i: (i, 0))


def _full_spec(r, c):
    return pl.BlockSpec((r, c), lambda i: (0, 0))


_ACC_SPEC = pl.BlockSpec((2, BN, H), lambda i: (0, i, 0))


def _tc_invc_body(cntp_r, out_r):
    out_r[:] = 1.0 / jnp.maximum(_col_sum(cntp_r[:]), 1.0)


def _tc_invc(cntp):
    return pl.pallas_call(
        _tc_invc_body,
        out_shape=jax.ShapeDtypeStruct((N, 1), jnp.float32),
    )(cntp)


def _tc_den_body(denp_r, out_r):
    out_r[:] = _col_sum(denp_r[:])


def _tc_den(denp):
    return pl.pallas_call(
        _tc_den_body,
        out_shape=jax.ShapeDtypeStruct((N, 1), jnp.float32),
    )(denp)


def _tc_build_body(dims_r, gidx_r, wdt_r, bd_r, emb_r, out_r):
    g = gidx_r[:]
    oh = (g == lax.broadcasted_iota(jnp.int32, (1, NUM_GATES), 1)).astype(jnp.float32)
    out_r[:, 0:H // 2] = dims_r[:] * wdt_r[:] + bd_r[:]
    out_r[:, H // 2:H] = jnp.dot(oh, emb_r[:], preferred_element_type=jnp.float32,
                                 precision=_HP)


def _tc_build(dims, gidx2, wdt, bd, emb):
    return pl.pallas_call(
        _tc_build_body,
        grid=(N // BN,),
        in_specs=[_rows_spec(1), _rows_spec(1), _full_spec(1, H // 2),
                  _full_spec(1, H // 2), _full_spec(NUM_GATES, H // 2)],
        out_specs=_rows_spec(H),
        out_shape=jax.ShapeDtypeStruct((N, H), jnp.float32),
    )(dims, gidx2, wdt, bd, emb)


def _sage_math(a, x, invc, wlt_r, bl_r, wrt_r):
    mean = a * invc
    z = (jnp.dot(mean, wlt_r[:], preferred_element_type=jnp.float32, precision=_HP)
         + bl_r[:]
         + jnp.dot(x, wrt_r[:], preferred_element_type=jnp.float32, precision=_HP))
    return jnp.maximum(z, 0.0)


def _tc_sage_body(acc_r, x_r, invc_r, wlt_r, bl_r, wrt_r, xout_r):
    a = acc_r[0] + acc_r[1]
    xout_r[:] = _sage_math(a, x_r[:], invc_r[:], wlt_r, bl_r, wrt_r)


def _tc_sage(acc3, x, invc, wlt, bl, wrt):
    return pl.pallas_call(
        _tc_sage_body,
        grid=(N // BN,),
        in_specs=[_ACC_SPEC, _rows_spec(H), _rows_spec(1), _full_spec(H, H),
                  _full_spec(1, H), _full_spec(H, H)],
        out_specs=_rows_spec(H),
        out_shape=jax.ShapeDtypeStruct((N, H), jnp.float32),
    )(acc3, x, invc, wlt, bl, wrt)


def _tc_gatprep_body(x_r, gwt_r, asc_r, adc_r, h_r, asr_r, adr_r):
    h = jnp.dot(x_r[:], gwt_r[:], preferred_element_type=jnp.float32, precision=_HP)
    h_r[:] = h
    asr_r[:] = jnp.dot(h, asc_r[:], preferred_element_type=jnp.float32, precision=_HP)
    adr_r[:] = jnp.dot(h, adc_r[:], preferred_element_type=jnp.float32, precision=_HP)


def _tc_gatprep(x, gwt, asc, adc):
    return pl.pallas_call(
        _tc_gatprep_body,
        grid=(N // BN,),
        in_specs=[_rows_spec(H), _full_spec(H, H), _full_spec(H, 1),
                  _full_spec(H, 1)],
        out_specs=(_rows_spec(H), _rows_spec(1), _rows_spec(1)),
        out_shape=(jax.ShapeDtypeStruct((N, H), jnp.float32),
                   jax.ShapeDtypeStruct((N, 1), jnp.float32),
                   jax.ShapeDtypeStruct((N, 1), jnp.float32)),
    )(x, gwt, asc, adc)


def _tc_gatfin_body(acc_r, den_r, h_r, asr_r, adr_r, gb_r, e0wt_r, e0bh_r, y_r):
    wh = acc_r[0] + acc_r[1]
    den = den_r[:]
    h = h_r[:]
    s = asr_r[:] + adr_r[:]
    wl = jnp.exp(jnp.maximum(s, 0.2 * s))
    x4 = jnp.maximum((wh + wl * h) / (den + wl + 1e-16) + gb_r[:], 0.0)
    y_r[:] = jnp.dot(x4, e0wt_r[:], preferred_element_type=jnp.float32,
                     precision=_HP) + e0bh_r[:]


def _tc_gatfin(acc3, den, h, asr, adr, gb, e0wt, e0bh):
    return pl.pallas_call(
        _tc_gatfin_body,
        grid=(N // BN,),
        in_specs=[_ACC_SPEC, _rows_spec(1), _rows_spec(H), _rows_spec(1),
                  _rows_spec(1), _full_spec(1, H), _full_spec(H, H),
                  _full_spec(1, H)],
        out_specs=_rows_spec(H),
        out_shape=jax.ShapeDtypeStruct((N, H), jnp.float32),
    )(acc3, den, h, asr, adr, gb, e0wt, e0bh)


BE = 2000


def _tc_final_body(z_r, e1wt_r, e1b_r, owt_r, ob_r, out_r):
    t = jnp.maximum(jnp.dot(z_r[:], e1wt_r[:], preferred_element_type=jnp.float32,
                            precision=_HP) + e1b_r[:], 0.0)
    out_r[:] = jnp.dot(t, owt_r[:], preferred_element_type=jnp.float32,
                       precision=_HP) + ob_r[:]


def _tc_final(z, e1wt, e1b, owt, ob):
    grid = (E // BE,)
    return pl.pallas_call(
        _tc_final_body,
        grid=grid,
        in_specs=[
            pl.BlockSpec((BE, H), lambda i: (i, 0)),
            pl.BlockSpec((H, H), lambda i: (0, 0)),
            pl.BlockSpec((1, H), lambda i: (0, 0)),
            pl.BlockSpec((H, 1), lambda i: (0, 0)),
            pl.BlockSpec((1, 1), lambda i: (0, 0)),
        ],
        out_specs=pl.BlockSpec((BE, 1), lambda i: (i, 0)),
        out_shape=jax.ShapeDtypeStruct((E, 1), jnp.float32),
    )(z, e1wt, e1b, owt, ob)


def kernel(dims, gate_indices, edge_index, emb_table, W_dim, b_dim,
           sage0_Wl, sage0_bl, sage0_Wr, sage1_Wl, sage1_bl, sage1_Wr,
           sage2_Wl, sage2_bl, sage2_Wr,
           gat_W, gat_att_src, gat_att_dst, gat_b,
           e0_W, e0_b, e1_W, e1_b, out_W, out_b):
    f32 = jnp.float32
    src = edge_index[0].astype(jnp.int32)
    dst = edge_index[1].astype(jnp.int32)
    gidx2 = gate_indices.astype(jnp.int32).reshape(N, 1)

    x0 = _tc_build(dims.astype(f32), gidx2, W_dim.reshape(1, H // 2),
                   b_dim.reshape(1, H // 2), emb_table)

    acc0, cntp = _sc_segsum_cnt(x0, src, dst)
    invc = _tc_invc(cntp.reshape(NW, N))
    x1 = _tc_sage(acc0.reshape(2, NPAD, H), x0, invc,
                  sage0_Wl.T, sage0_bl.reshape(1, H), sage0_Wr.T)

    acc1 = _sc_segsum(x1, src, dst)
    x2 = _tc_sage(acc1.reshape(2, NPAD, H), x1, invc,
                  sage1_Wl.T, sage1_bl.reshape(1, H), sage1_Wr.T)

    acc2 = _sc_segsum(x2, src, dst)
    x3 = _tc_sage(acc2.reshape(2, NPAD, H), x2, invc,
                  sage2_Wl.T, sage2_bl.reshape(1, H), sage2_Wr.T)

    h, asr, adr = _tc_gatprep(x3, gat_W.T, gat_att_src.reshape(H, 1),
                              gat_att_dst.reshape(H, 1))
    w, denp = _sc_gatw(asr.reshape(N), adr.reshape(N), src, dst)
    accg = _sc_gat(h, w, src, dst)
    den = _tc_den(denp.reshape(NW, N))
    y = _tc_gatfin(accg.reshape(2, NPAD, H), den, h, asr, adr,
                   gat_b.reshape(1, H), e0_W.T, (0.5 * e0_b).reshape(1, H))

    z = _sc_edge(y, src, dst)
    return _tc_final(z, e1_W.T, e1_b.reshape(1, H), out_W.T, out_b.reshape(1, 1))


# speedup vs baseline: 10.7580x; 1.2127x over previous
import functools

import jax
import jax.numpy as jnp
from jax import lax
from jax.experimental import pallas as pl
from jax.experimental.pallas import tpu as pltpu
from jax.experimental.pallas import tpu_sc as plsc

N = 10000
E = 320000
H = 128
NUM_GATES = 9

NC = 2
NS = 16
NW = NC * NS
EPW = E // NW
C = 80
NCHUNK = EPW // C
NB = 2
NITER = (NCHUNK + NB - 1) // NB
NPAD = 10240
RPT = NPAD // NS
ZR = 80

_mesh = plsc.VectorSubcoreMesh(core_axis_name="c", subcore_axis_name="s",
                               num_cores=NC, num_subcores=NS)
_sc_params = pltpu.CompilerParams(needs_layout_passes=False)


def _zero_rows(rows):
    def zrow(r, _):
        for c in range(H // 16):
            rows[r, pl.ds(c * 16, 16)] = jnp.zeros((16,), jnp.float32)
        return 0
    lax.fori_loop(0, ZR, zrow, 0)


def _zero_vec(v):
    def zrow(r, _):
        v[pl.ds(r * 16, 16)] = jnp.zeros((16,), jnp.float32)
        return 0
    lax.fori_loop(0, N // 16, zrow, 0)


def _zero_accum_slice(rows, accum, sid):
    def zcopy(j, _):
        pltpu.sync_copy(rows.at[pl.ds(0, ZR)], accum.at[pl.ds(sid * RPT + j * ZR, ZR)])
        return 0
    lax.fori_loop(0, RPT // ZR, zcopy, 0)


def _write_out(rows, accum, out_hbm, cid, sid):
    def ocopy(j, _):
        pltpu.sync_copy(accum.at[pl.ds(sid * RPT + j * ZR, ZR)], rows.at[pl.ds(0, ZR)])
        pltpu.sync_copy(rows.at[pl.ds(0, ZR)],
                        out_hbm.at[pl.ds(cid * NPAD + sid * RPT + j * ZR, ZR)])
        return 0
    lax.fori_loop(0, RPT // ZR, ocopy, 0)


def _segsum_pipeline(x_hbm, src_hbm, dst_hbm, accum, ebase,
                     sidx, didx, rows, gsem, ssem, on_chunk=None):
    def prep(k, b):
        base = ebase + k * C
        pltpu.sync_copy(src_hbm.at[pl.ds(base, C)], sidx[b])
        pltpu.sync_copy(dst_hbm.at[pl.ds(base, C)], didx[b])
        pltpu.async_copy(x_hbm.at[sidx[b]], rows[b], gsem[b])

    for b in range(NB):
        prep(b, b)

    def outer(g, _):
        for b in range(NB):
            k = NB * g + b

            @pl.when(k < NCHUNK)
            def _step(b=b, k=k):
                pltpu.make_async_copy(x_hbm.at[sidx[b]], rows[b], gsem[b]).wait()
                if on_chunk is not None:
                    on_chunk(k, b)
                pltpu.async_copy(rows[b], accum.at[didx[b]], ssem[b], add=True)

                @pl.when(k + NB < NCHUNK)
                def _prep_next(b=b, k=k):
                    pltpu.make_async_copy(rows[b], accum.at[didx[b]], ssem[b]).wait()
                    prep(k + NB, b)
        return 0
    lax.fori_loop(0, NITER, outer, 0)
    for b in range(NB):
        pltpu.make_async_copy(rows[b], accum.at[didx[b]], ssem[b]).wait()


_SEG_SCRATCH = [
    pltpu.VMEM((C,), jnp.int32), pltpu.VMEM((C,), jnp.int32),
    pltpu.VMEM((C,), jnp.int32), pltpu.VMEM((C,), jnp.int32),
    pltpu.VMEM((C, H), jnp.float32), pltpu.VMEM((C, H), jnp.float32),
    pltpu.VMEM_SHARED((NPAD, H), jnp.float32),
    pltpu.SemaphoreType.DMA, pltpu.SemaphoreType.DMA,
    pltpu.SemaphoreType.DMA, pltpu.SemaphoreType.DMA,
]


@functools.partial(
    pl.kernel, mesh=_mesh, compiler_params=_sc_params,
    out_type=(jax.ShapeDtypeStruct((2 * NPAD, H), jnp.float32),
              jax.ShapeDtypeStruct((NW * N,), jnp.float32)),
    scratch_types=_SEG_SCRATCH + [pltpu.VMEM((N,), jnp.float32)],
)
def _sc_segsum_cnt(x_hbm, src_hbm, dst_hbm, out_hbm, cnt_hbm,
                   sidx0, sidx1, didx0, didx1, rows0, rows1, accum,
                   gsem0, gsem1, ssem0, ssem1, cnt_v):
    cid = lax.axis_index("c")
    sid = lax.axis_index("s")
    wid = sid * NC + cid
    _zero_rows(rows0)
    _zero_accum_slice(rows0, accum, sid)
    _zero_vec(cnt_v)
    plsc.subcore_barrier()
    didx = (didx0, didx1)
    ones16 = jnp.full((16,), 1.0, jnp.float32)

    def on_chunk(k, b):
        def cgrp(j, _):
            jv = didx[b][pl.ds(j * 16, 16)]
            plsc.addupdate_scatter(cnt_v, [jv], ones16)
            return 0
        lax.fori_loop(0, C // 16, cgrp, 0)

    _segsum_pipeline(x_hbm, src_hbm, dst_hbm, accum, wid * EPW,
                     (sidx0, sidx1), didx, (rows0, rows1),
                     (gsem0, gsem1), (ssem0, ssem1), on_chunk)
    plsc.subcore_barrier()
    _write_out(rows0, accum, out_hbm, cid, sid)
    pltpu.sync_copy(cnt_v, cnt_hbm.at[pl.ds(wid * N, N)])


@functools.partial(
    pl.kernel, mesh=_mesh, compiler_params=_sc_params,
    out_type=jax.ShapeDtypeStruct((2 * NPAD, H), jnp.float32),
    scratch_types=_SEG_SCRATCH,
)
def _sc_segsum(x_hbm, src_hbm, dst_hbm, out_hbm,
               sidx0, sidx1, didx0, didx1, rows0, rows1, accum,
               gsem0, gsem1, ssem0, ssem1):
    cid = lax.axis_index("c")
    sid = lax.axis_index("s")
    wid = sid * NC + cid
    _zero_rows(rows0)
    _zero_accum_slice(rows0, accum, sid)
    plsc.subcore_barrier()
    _segsum_pipeline(x_hbm, src_hbm, dst_hbm, accum, wid * EPW,
                     (sidx0, sidx1), (didx0, didx1), (rows0, rows1),
                     (gsem0, gsem1), (ssem0, ssem1))
    plsc.subcore_barrier()
    _write_out(rows0, accum, out_hbm, cid, sid)


@functools.partial(
    pl.kernel, mesh=_mesh, compiler_params=_sc_params,
    out_type=(jax.ShapeDtypeStruct((E,), jnp.float32),
              jax.ShapeDtypeStruct((NW * N,), jnp.float32)),
    scratch_types=[
        pltpu.VMEM((EPW,), jnp.int32),
        pltpu.VMEM((EPW,), jnp.int32),
        pltpu.VMEM((EPW,), jnp.float32),
        pltpu.VMEM((N,), jnp.float32),
        pltpu.VMEM((N,), jnp.float32),
        pltpu.VMEM((N,), jnp.float32),
    ],
)
def _sc_gatw(asrc_hbm, adst_hbm, src_hbm, dst_hbm, w_hbm, den_hbm,
             sidx_all, didx_all, wall, asv, adv, den_v):
    cid = lax.axis_index("c")
    sid = lax.axis_index("s")
    wid = sid * NC + cid
    ebase = wid * EPW
    pltpu.sync_copy(src_hbm.at[pl.ds(ebase, EPW)], sidx_all)
    pltpu.sync_copy(dst_hbm.at[pl.ds(ebase, EPW)], didx_all)
    pltpu.sync_copy(asrc_hbm, asv)
    pltpu.sync_copy(adst_hbm, adv)
    _zero_vec(den_v)

    def grp(j, _):
        sl = pl.ds(j * 16, 16)
        iv = sidx_all[sl]
        jv = didx_all[sl]
        s = plsc.load_gather(asv, [iv]) + plsc.load_gather(adv, [jv])
        w = jnp.exp(jnp.maximum(s, 0.2 * s))
        wall[sl] = w
        plsc.addupdate_scatter(den_v, [jv], w)
        return 0
    lax.fori_loop(0, EPW // 16, grp, 0)
    pltpu.sync_copy(wall, w_hbm.at[pl.ds(ebase, EPW)])
    pltpu.sync_copy(den_v, den_hbm.at[pl.ds(wid * N, N)])


@functools.partial(
    pl.kernel, mesh=_mesh, compiler_params=_sc_params,
    out_type=jax.ShapeDtypeStruct((2 * NPAD, H), jnp.float32),
    scratch_types=_SEG_SCRATCH + [pltpu.VMEM((C,), jnp.float32),
                                  pltpu.VMEM((C,), jnp.float32)],
)
def _sc_gat(h_hbm, w_hbm, src_hbm, dst_hbm, out_hbm,
            sidx0, sidx1, didx0, didx1, rows0, rows1, accum,
            gsem0, gsem1, ssem0, ssem1, wc0, wc1):
    cid = lax.axis_index("c")
    sid = lax.axis_index("s")
    wid = sid * NC + cid
    ebase = wid * EPW
    _zero_rows(rows0)
    _zero_accum_slice(rows0, accum, sid)
    plsc.subcore_barrier()
    rows = (rows0, rows1)
    wc = (wc0, wc1)

    def on_chunk(k, b):
        pltpu.sync_copy(w_hbm.at[pl.ds(ebase + k * C, C)], wc[b])

        def scale(r, _):
            wr = plsc.load_gather(wc[b], [jnp.zeros((16,), jnp.int32) + r])
            for c in range(H // 16):
                sl = pl.ds(c * 16, 16)
                rows[b][r, sl] = rows[b][r, sl] * wr
            return 0
        lax.fori_loop(0, C, scale, 0)

    _segsum_pipeline(h_hbm, src_hbm, dst_hbm, accum, ebase,
                     (sidx0, sidx1), (didx0, didx1), rows,
                     (gsem0, gsem1), (ssem0, ssem1), on_chunk)
    plsc.subcore_barrier()
    _write_out(rows0, accum, out_hbm, cid, sid)


@functools.partial(
    pl.kernel, mesh=_mesh, compiler_params=_sc_params,
    out_type=jax.ShapeDtypeStruct((E, H), jnp.float32),
    scratch_types=[
        pltpu.VMEM((C,), jnp.int32), pltpu.VMEM((C,), jnp.int32),
        pltpu.VMEM((C,), jnp.int32), pltpu.VMEM((C,), jnp.int32),
        pltpu.VMEM((C, H), jnp.float32), pltpu.VMEM((C, H), jnp.float32),
        pltpu.VMEM((C, H), jnp.float32), pltpu.VMEM((C, H), jnp.float32),
        pltpu.SemaphoreType.DMA, pltpu.SemaphoreType.DMA,
        pltpu.SemaphoreType.DMA, pltpu.SemaphoreType.DMA,
        pltpu.SemaphoreType.DMA, pltpu.SemaphoreType.DMA,
    ],
)
def _sc_edge(y_hbm, src_hbm, dst_hbm, z_hbm,
             sidx0, sidx1, didx0, didx1, rs0, rs1, rd0, rd1,
             gs0, gs1, gd0, gd1, zsem0, zsem1):
    cid = lax.axis_index("c")
    sid = lax.axis_index("s")
    wid = sid * NC + cid
    ebase = wid * EPW
    sidx = (sidx0, sidx1)
    didx = (didx0, didx1)
    rs = (rs0, rs1)
    rd = (rd0, rd1)
    gs = (gs0, gs1)
    gd = (gd0, gd1)
    zsem = (zsem0, zsem1)

    def prep(k, b):
        base = ebase + k * C
        pltpu.sync_copy(src_hbm.at[pl.ds(base, C)], sidx[b])
        pltpu.sync_copy(dst_hbm.at[pl.ds(base, C)], didx[b])
        pltpu.async_copy(y_hbm.at[sidx[b]], rs[b], gs[b])
        pltpu.async_copy(y_hbm.at[didx[b]], rd[b], gd[b])

    for b in range(NB):
        prep(b, b)

    def outer(g, _):
        for b in range(NB):
            k = NB * g + b

            @pl.when(k < NCHUNK)
            def _step(b=b, k=k):
                base = ebase + k * C
                pltpu.make_async_copy(y_hbm.at[sidx[b]], rs[b], gs[b]).wait()
                pltpu.make_async_copy(y_hbm.at[didx[b]], rd[b], gd[b]).wait()

                def add(r, _):
                    for c in range(H // 16):
                        sl = pl.ds(c * 16, 16)
                        rs[b][r, sl] = rs[b][r, sl] + rd[b][r, sl]
                    return 0
                lax.fori_loop(0, C, add, 0)
                pltpu.async_copy(rs[b], z_hbm.at[pl.ds(base, C)], zsem[b])

                @pl.when(k + NB < NCHUNK)
                def _prep_next(b=b, k=k):
                    pltpu.make_async_copy(rs[b], z_hbm.at[pl.ds(ebase, C)],
                                          zsem[b]).wait()
                    prep(k + NB, b)
        return 0
    lax.fori_loop(0, NITER, outer, 0)
    for b in range(NB):
        pltpu.make_async_copy(rs[b], z_hbm.at[pl.ds(ebase, C)], zsem[b]).wait()


BN = 2000
_HP = lax.Precision.HIGHEST


def _col_sum(p2d):
    ones = jnp.full((NW, 1), 1.0, jnp.float32)
    return lax.dot_general(p2d, ones, (((0,), (0,)), ((), ())),
                           preferred_element_type=jnp.float32, precision=_HP)


def _rows_spec(width):
    return pl.BlockSpec((BN, width), lambda i: (i, 0))


def _full_spec(r, c):
    return pl.BlockSpec((r, c), lambda i: (0, 0))


_ACC_SPEC = pl.BlockSpec((2, BN, H), lambda i: (0, i, 0))


def _tc_cnt_body(cntp_r, out_r):
    out_r[:] = _col_sum(cntp_r[:])


def _tc_cnt(cntp):
    return pl.pallas_call(
        _tc_cnt_body,
        out_shape=jax.ShapeDtypeStruct((N, 1), jnp.float32),
    )(cntp)


def _tc_den_body(denp_r, out_r):
    out_r[:] = _col_sum(denp_r[:])


def _tc_den(denp):
    return pl.pallas_call(
        _tc_den_body,
        out_shape=jax.ShapeDtypeStruct((N, 1), jnp.float32),
    )(denp)


def _tc_build_body(dims_r, gidx_r, wdt_r, bd_r, emb_r, out_r):
    g = gidx_r[:]
    oh = (g == lax.broadcasted_iota(jnp.int32, (1, NUM_GATES), 1)).astype(jnp.float32)
    out_r[:, 0:H // 2] = dims_r[:] * wdt_r[:] + bd_r[:]
    out_r[:, H // 2:H] = jnp.dot(oh, emb_r[:], preferred_element_type=jnp.float32,
                                 precision=_HP)


def _tc_build(dims, gidx2, wdt, bd, emb):
    return pl.pallas_call(
        _tc_build_body,
        grid=(N // BN,),
        in_specs=[_rows_spec(1), _rows_spec(1), _full_spec(1, H // 2),
                  _full_spec(1, H // 2), _full_spec(NUM_GATES, H // 2)],
        out_specs=_rows_spec(H),
        out_shape=jax.ShapeDtypeStruct((N, H), jnp.float32),
    )(dims, gidx2, wdt, bd, emb)


def _sage_math(a, x, cnt, wlt_r, bl_r, wrt_r):
    mean = a / jnp.maximum(cnt, 1.0)
    z = (jnp.dot(mean, wlt_r[:], preferred_element_type=jnp.float32)
         + bl_r[:]
         + jnp.dot(x, wrt_r[:], preferred_element_type=jnp.float32))
    return jnp.maximum(z, 0.0)


def _tc_sage_body(acc_r, x_r, cnt_r, wlt_r, bl_r, wrt_r, xout_r):
    a = acc_r[0] + acc_r[1]
    xout_r[:] = _sage_math(a, x_r[:], cnt_r[:], wlt_r, bl_r, wrt_r)


def _tc_sage(acc3, x, cnt, wlt, bl, wrt):
    return pl.pallas_call(
        _tc_sage_body,
        grid=(N // BN,),
        in_specs=[_ACC_SPEC, _rows_spec(H), _rows_spec(1), _full_spec(H, H),
                  _full_spec(1, H), _full_spec(H, H)],
        out_specs=_rows_spec(H),
        out_shape=jax.ShapeDtypeStruct((N, H), jnp.float32),
    )(acc3, x, cnt, wlt, bl, wrt)


def _tc_gatprep_body(x_r, gwt_r, asc_r, adc_r, h_r, asr_r, adr_r):
    h = jnp.dot(x_r[:], gwt_r[:], preferred_element_type=jnp.float32)
    h_r[:] = h
    asr_r[:] = jnp.dot(h, asc_r[:], preferred_element_type=jnp.float32)
    adr_r[:] = jnp.dot(h, adc_r[:], preferred_element_type=jnp.float32)


def _tc_gatprep(x, gwt, asc, adc):
    return pl.pallas_call(
        _tc_gatprep_body,
        grid=(N // BN,),
        in_specs=[_rows_spec(H), _full_spec(H, H), _full_spec(H, 1),
                  _full_spec(H, 1)],
        out_specs=(_rows_spec(H), _rows_spec(1), _rows_spec(1)),
        out_shape=(jax.ShapeDtypeStruct((N, H), jnp.float32),
                   jax.ShapeDtypeStruct((N, 1), jnp.float32),
                   jax.ShapeDtypeStruct((N, 1), jnp.float32)),
    )(x, gwt, asc, adc)


def _tc_gatfin_body(acc_r, den_r, h_r, asr_r, adr_r, gb_r, x4_r):
    wh = acc_r[0] + acc_r[1]
    den = den_r[:]
    h = h_r[:]
    s = asr_r[:] + adr_r[:]
    wl = jnp.exp(jnp.maximum(s, 0.2 * s))
    x4_r[:] = jnp.maximum((wh + wl * h) / (den + wl + 1e-16) + gb_r[:], 0.0)


def _tc_gatfin(acc3, den, h, asr, adr, gb):
    return pl.pallas_call(
        _tc_gatfin_body,
        grid=(N // BN,),
        in_specs=[_ACC_SPEC, _rows_spec(1), _rows_spec(H), _rows_spec(1),
                  _rows_spec(1), _full_spec(1, H)],
        out_specs=_rows_spec(H),
        out_shape=jax.ShapeDtypeStruct((N, H), jnp.float32),
    )(acc3, den, h, asr, adr, gb)


BE = 2000


def _tc_final_body(xe_r, e0wt_r, e0b_r, e1wt_r, e1b_r, owt_r, ob_r, out_r):
    t0 = jnp.maximum(jnp.dot(xe_r[:], e0wt_r[:],
                             preferred_element_type=jnp.float32) + e0b_r[:], 0.0)
    t1 = jnp.maximum(jnp.dot(t0, e1wt_r[:],
                             preferred_element_type=jnp.float32) + e1b_r[:], 0.0)
    out_r[:] = jnp.dot(t1, owt_r[:], preferred_element_type=jnp.float32) + ob_r[:]


def _tc_final(xe, e0wt, e0b, e1wt, e1b, owt, ob):
    grid = (E // BE,)
    return pl.pallas_call(
        _tc_final_body,
        grid=grid,
        in_specs=[
            pl.BlockSpec((BE, H), lambda i: (i, 0)),
            pl.BlockSpec((H, H), lambda i: (0, 0)),
            pl.BlockSpec((1, H), lambda i: (0, 0)),
            pl.BlockSpec((H, H), lambda i: (0, 0)),
            pl.BlockSpec((1, H), lambda i: (0, 0)),
            pl.BlockSpec((H, 1), lambda i: (0, 0)),
            pl.BlockSpec((1, 1), lambda i: (0, 0)),
        ],
        out_specs=pl.BlockSpec((BE, 1), lambda i: (i, 0)),
        out_shape=jax.ShapeDtypeStruct((E, 1), jnp.float32),
    )(xe, e0wt, e0b, e1wt, e1b, owt, ob)


def kernel(dims, gate_indices, edge_index, emb_table, W_dim, b_dim,
           sage0_Wl, sage0_bl, sage0_Wr, sage1_Wl, sage1_bl, sage1_Wr,
           sage2_Wl, sage2_bl, sage2_Wr,
           gat_W, gat_att_src, gat_att_dst, gat_b,
           e0_W, e0_b, e1_W, e1_b, out_W, out_b):
    f32 = jnp.float32
    src = edge_index[0].astype(jnp.int32)
    dst = edge_index[1].astype(jnp.int32)
    gidx2 = gate_indices.astype(jnp.int32).reshape(N, 1)

    x0 = _tc_build(dims.astype(f32), gidx2, W_dim.reshape(1, H // 2),
                   b_dim.reshape(1, H // 2), emb_table)

    acc0, cntp = _sc_segsum_cnt(x0, src, dst)
    cnt = _tc_cnt(cntp.reshape(NW, N))
    x1 = _tc_sage(acc0.reshape(2, NPAD, H), x0, cnt,
                  sage0_Wl.T, sage0_bl.reshape(1, H), sage0_Wr.T)

    acc1 = _sc_segsum(x1, src, dst)
    x2 = _tc_sage(acc1.reshape(2, NPAD, H), x1, cnt,
                  sage1_Wl.T, sage1_bl.reshape(1, H), sage1_Wr.T)

    acc2 = _sc_segsum(x2, src, dst)
    x3 = _tc_sage(acc2.reshape(2, NPAD, H), x2, cnt,
                  sage2_Wl.T, sage2_bl.reshape(1, H), sage2_Wr.T)

    h, asr, adr = _tc_gatprep(x3, gat_W.T, gat_att_src.reshape(H, 1),
                              gat_att_dst.reshape(H, 1))
    w, denp = _sc_gatw(asr.reshape(N), adr.reshape(N), src, dst)
    accg = _sc_gat(h, w, src, dst)
    den = _tc_den(denp.reshape(NW, N))
    x4 = _tc_gatfin(accg.reshape(2, NPAD, H), den, h, asr, adr,
                    gat_b.reshape(1, H))

    xe = _sc_edge(x4, src, dst)
    return _tc_final(xe, e0_W.T, e0_b.reshape(1, H), e1_W.T, e1_b.reshape(1, H),
                     out_W.T, out_b.reshape(1, 1))
